# Initial kernel scaffold; baseline (speedup 1.0000x reference)
import jax
import jax.numpy as jnp
from jax.experimental import pallas as pl


def kernel(x, W_enc, b_enc, W_dec, b_dec):
    raise NotImplementedError("write your pallas kernel here")


# speedup vs baseline: 11.2321x; 11.2321x over previous
import jax
import jax.numpy as jnp
from jax import lax
from jax.experimental import pallas as pl
from jax.experimental.pallas import tpu as pltpu

K = 40
ROWS_A = 128
ROWS_B = 128


def _encode_body(x_ref, wenc_ref, benc_ref, bdec_ref, codes_ref):
    xin = x_ref[...] - bdec_ref[...]
    pre = jnp.dot(xin, wenc_ref[...], preferred_element_type=jnp.float32)
    a = jnp.maximum(pre + benc_ref[...], 0.0)
    rows = a.shape[0]

    def it(_, carry):
        lo, hi = carry
        mid = lo + (hi - lo) // 2
        t = lax.bitcast_convert_type(mid, jnp.float32)
        cnt = jnp.sum((a >= t).astype(jnp.int32), axis=1, keepdims=True)
        ge = cnt >= K
        return jnp.where(ge, mid, lo), jnp.where(ge, hi, mid)

    lo0 = jnp.zeros((rows, 1), jnp.int32)
    hi0 = jnp.full((rows, 1), jnp.int32(0x7F800000))
    lo, _ = lax.fori_loop(0, 31, it, (lo0, hi0))
    thr = lax.bitcast_convert_type(lo, jnp.float32)
    codes_ref[...] = jnp.where(a >= thr, a, 0.0)


def _decode_body(codes_ref, wdec_ref, bdec_ref, recon_ref):
    recon_ref[...] = (
        jnp.dot(codes_ref[...], wdec_ref[...], preferred_element_type=jnp.float32)
        + bdec_ref[...]
    )


@jax.jit
def kernel(x, W_enc, b_enc, W_dec, b_dec):
    B, d_in = x.shape
    d_sae = W_enc.shape[1]

    codes = pl.pallas_call(
        _encode_body,
        grid=(B // ROWS_A,),
        in_specs=[
            pl.BlockSpec((ROWS_A, d_in), lambda i: (i, 0)),
            pl.BlockSpec((d_in, d_sae), lambda i: (0, 0)),
            pl.BlockSpec((1, d_sae), lambda i: (0, 0)),
            pl.BlockSpec((1, d_in), lambda i: (0, 0)),
        ],
        out_specs=pl.BlockSpec((ROWS_A, d_sae), lambda i: (i, 0)),
        out_shape=jax.ShapeDtypeStruct((B, d_sae), jnp.float32),
        compiler_params=pltpu.CompilerParams(
            vmem_limit_bytes=64 * 1024 * 1024,
        ),
    )(x, W_enc, b_enc.reshape(1, d_sae), b_dec.reshape(1, d_in))

    recon = pl.pallas_call(
        _decode_body,
        grid=(B // ROWS_B,),
        in_specs=[
            pl.BlockSpec((ROWS_B, d_sae), lambda i: (i, 0)),
            pl.BlockSpec((d_sae, d_in), lambda i: (0, 0)),
            pl.BlockSpec((1, d_in), lambda i: (0, 0)),
        ],
        out_specs=pl.BlockSpec((ROWS_B, d_in), lambda i: (i, 0)),
        out_shape=jax.ShapeDtypeStruct((B, d_in), jnp.float32),
        compiler_params=pltpu.CompilerParams(
            vmem_limit_bytes=64 * 1024 * 1024,
        ),
    )(codes, W_dec, b_dec.reshape(1, d_in))

    return recon, codes
